# baseline (speedup 1.0000x reference)
import functools

import jax
import jax.numpy as jnp
from jax import lax
from jax.experimental import pallas as pl
from jax.experimental.pallas import tpu as pltpu
from jax.experimental.pallas import tpu_sc as plsc

B, S = 64, 512
H = 768
P, T = 512, 2
TOK = B * S
NC, NS, L = 2, 16, 16
NW = NC * NS
TPW = TOK // NW
C = 64
NCH = TPW // C
NSL = H // L
EPS = 1e-12
INV_H = 1.0 / H


_GATHER_DNUMS = lax.GatherDimensionNumbers(
    offset_dims=(), collapsed_slice_dims=(0,), start_index_map=(0,))


def _lane_gather(x, idx):
    return lax.gather(x, idx[:, None], _GATHER_DNUMS, (1,),
                      mode=lax.GatherScatterMode.PROMISE_IN_BOUNDS)


def _allsum(x):
    for sh in (8, 4, 2, 1):
        idx = lax.iota(jnp.int32, L) ^ sh
        x = x + _lane_gather(x, idx)
    return x


def _ln_token(t, word_v, bias_v, gamma_v, beta_v):
    acc_s = jnp.zeros((L,), jnp.float32)
    acc_q = jnp.zeros((L,), jnp.float32)
    for j in range(NSL):
        sl = pl.ds(j * L, L)
        x = word_v[t, sl] + bias_v[t, sl]
        word_v[t, sl] = x
        acc_s = acc_s + x
        acc_q = acc_q + x * x
    mean_v = _allsum(acc_s) * INV_H
    var_v = _allsum(acc_q) * INV_H - mean_v * mean_v
    vv = var_v + EPS
    iv = plsc.bitcast(vv, jnp.int32)
    yi = jnp.int32(0x5F3759DF) - lax.shift_right_logical(iv, 1)
    y = plsc.bitcast(yi, jnp.float32)
    for _ in range(3):
        y = y * (1.5 - 0.5 * vv * y * y)
    for j in range(NSL):
        sl = pl.ds(j * L, L)
        x = word_v[t, sl]
        word_v[t, sl] = (x - mean_v) * y * gamma_v[sl] + beta_v[sl]
    return 0


def _body(ids_hbm, tids_hbm, word_hbm, comb_hbm, gamma_hbm, beta_hbm, out_hbm,
          idx_v, tid_v, idx2_v, word_v, bias_v, gamma_v, beta_v, sem_w, sem_b):
    wid = lax.axis_index("s") * NC + lax.axis_index("c")
    base = wid * TPW
    pltpu.sync_copy(gamma_hbm, gamma_v)
    pltpu.sync_copy(beta_hbm, beta_v)

    def chunk(ci, carry):
        g0 = base + ci * C
        pltpu.sync_copy(ids_hbm.at[pl.ds(g0, C)], idx_v)
        pltpu.sync_copy(tids_hbm.at[pl.ds(g0, C)], tid_v)
        s0 = lax.rem(g0, S)
        for g in range(C // L):
            tv = tid_v[pl.ds(g * L, L)]
            sv = lax.iota(jnp.int32, L) + (s0 + g * L)
            idx2_v[pl.ds(g * L, L)] = tv * S + sv
        cw = pltpu.async_copy(word_hbm.at[idx_v], word_v, sem_w)
        cb = pltpu.async_copy(comb_hbm.at[idx2_v], bias_v, sem_b)
        cw.wait()
        cb.wait()
        lax.fori_loop(
            0, C, lambda t, c: _ln_token(t, word_v, bias_v, gamma_v, beta_v), 0)
        pltpu.sync_copy(word_v, out_hbm.at[pl.ds(g0, C)])
        return carry

    lax.fori_loop(0, NCH, chunk, 0)


@functools.partial(jax.jit, static_argnames=())
def _emb(ids, tids, word_table, comb, gamma, beta):
    mesh = plsc.VectorSubcoreMesh(core_axis_name="c", subcore_axis_name="s")
    f = pl.kernel(
        _body,
        out_type=jax.ShapeDtypeStruct((TOK, H), jnp.float32),
        mesh=mesh,
        compiler_params=pltpu.CompilerParams(needs_layout_passes=False),
        scratch_types=[
            pltpu.VMEM((C,), jnp.int32),
            pltpu.VMEM((C,), jnp.int32),
            pltpu.VMEM((C,), jnp.int32),
            pltpu.VMEM((C, H), jnp.float32),
            pltpu.VMEM((C, H), jnp.float32),
            pltpu.VMEM((H,), jnp.float32),
            pltpu.VMEM((H,), jnp.float32),
            pltpu.SemaphoreType.DMA,
            pltpu.SemaphoreType.DMA,
        ],
    )
    return f(ids, tids, word_table, comb, gamma, beta)


def kernel(input_ids, token_type_ids, word_table, pos_table, type_table, gamma, beta):
    ids = input_ids.reshape(-1).astype(jnp.int32)
    tids = token_type_ids.reshape(-1).astype(jnp.int32)
    comb = (type_table[:, None, :] + pos_table[None, :, :]).reshape(T * P, H)
    out = _emb(ids, tids, word_table, comb, gamma, beta)
    return out.reshape(input_ids.shape[0], input_ids.shape[1], H)
